# baseline (speedup 1.0000x reference)
import functools

import jax
import jax.numpy as jnp
from jax import lax
from jax.experimental import pallas as pl
from jax.experimental.pallas import tpu as pltpu

K = 2
SUB = 256
BF = 256


def _gate_up_body(pstart_ref, counts_ref, x_ref, w1_ref, w3_ref, h_ref):
    e = pl.program_id(1)
    p0 = pstart_ref[e]
    cnt = counts_ref[e]
    nsub = (cnt + SUB - 1) // SUB
    w1 = w1_ref[0]
    w3 = w3_ref[0]

    def body(j, carry):
        off = pl.multiple_of(p0 + j * SUB, SUB)
        x = x_ref[pl.ds(off, SUB), :]
        gate = lax.dot_general(x, w1, (((1,), (1,)), ((), ())),
                               preferred_element_type=jnp.float32,
                               precision=lax.Precision.DEFAULT)
        up = lax.dot_general(x, w3, (((1,), (1,)), ((), ())),
                             preferred_element_type=jnp.float32,
                             precision=lax.Precision.DEFAULT)
        h = gate * jax.nn.sigmoid(gate) * up
        h_ref[pl.ds(off, SUB), :] = h.astype(jnp.bfloat16)
        return carry

    lax.fori_loop(0, nsub, body, 0)


def _down_body(be_ref, act_ref, h_ref, w2_ref, y_ref):
    g = pl.program_id(0)

    @pl.when(act_ref[g] > 0)
    def _():
        h = h_ref[...].astype(jnp.float32)
        y_ref[...] = lax.dot_general(
            h, w2_ref[0], (((1,), (1,)), ((), ())),
            preferred_element_type=jnp.float32,
            precision=lax.Precision.DEFAULT)


def _grouped_mlp(x_sorted, w13, w2, pstart, counts, blk_expert, blk_active,
                 n_experts, d_ff):
    m_pad, d_model = x_sorted.shape
    nf = d_ff // BF
    n_blocks = m_pad // SUB

    gu_spec = pltpu.PrefetchScalarGridSpec(
        num_scalar_prefetch=2,
        grid=(nf, n_experts),
        in_specs=[
            pl.BlockSpec((m_pad, d_model), lambda f, e, ps, ct: (0, 0)),
            pl.BlockSpec((1, BF, d_model), lambda f, e, ps, ct: (e, f, 0)),
            pl.BlockSpec((1, BF, d_model), lambda f, e, ps, ct: (e, nf + f, 0)),
        ],
        out_specs=pl.BlockSpec((m_pad, BF), lambda f, e, ps, ct: (0, f)),
    )
    h = pl.pallas_call(
        _gate_up_body,
        grid_spec=gu_spec,
        out_shape=jax.ShapeDtypeStruct((m_pad, d_ff), jnp.bfloat16),
        compiler_params=pltpu.CompilerParams(
            dimension_semantics=("arbitrary", "arbitrary"),
        ),
    )(pstart, counts, x_sorted, w13, w13)

    dn_spec = pltpu.PrefetchScalarGridSpec(
        num_scalar_prefetch=2,
        grid=(n_blocks,),
        in_specs=[
            pl.BlockSpec((SUB, d_ff), lambda g, be, act: (g, 0)),
            pl.BlockSpec((1, d_model, d_ff), lambda g, be, act: (be[g], 0, 0)),
        ],
        out_specs=pl.BlockSpec((SUB, d_model), lambda g, be, act: (g, 0)),
    )
    return pl.pallas_call(
        _down_body,
        grid_spec=dn_spec,
        out_shape=jax.ShapeDtypeStruct((m_pad, d_model), jnp.float32),
        compiler_params=pltpu.CompilerParams(
            dimension_semantics=("arbitrary",),
        ),
    )(blk_expert, blk_active, h, w2)


def kernel(hidden_states, use_grouped_topk, top_k, router_logits,
           renormalize, W13, W2):
    t, d_model = hidden_states.shape
    e = router_logits.shape[1]
    d_ff = W2.shape[2]
    m = t * K
    m_pad = m + e * SUB
    n_blocks = m_pad // SUB
    iota_e = jnp.arange(e, dtype=jnp.int32)

    logits = router_logits.astype(jnp.float32)
    mx = jnp.max(logits, axis=-1, keepdims=True)
    ex = jnp.exp(logits - mx)
    probs = ex / jnp.sum(ex, axis=-1, keepdims=True)
    m1 = jnp.max(probs, axis=-1)
    am1 = jnp.min(jnp.where(probs == m1[:, None], iota_e, e),
                  axis=-1).astype(jnp.int32)
    masked = jnp.where(iota_e[None, :] == am1[:, None], -jnp.inf, probs)
    m2 = jnp.max(masked, axis=-1)
    am2 = jnp.min(jnp.where(masked == m2[:, None], iota_e, e),
                  axis=-1).astype(jnp.int32)
    topk_w = jnp.stack([m1, m2], axis=1)
    denom = m1 + m2
    topk_w = jnp.where(jnp.asarray(renormalize), topk_w / denom[:, None],
                       topk_w)
    topk_w = topk_w * (jnp.asarray(1, jnp.float32)
                       - jnp.asarray(use_grouped_topk, jnp.float32))

    oh1 = (am1[:, None] == iota_e[None, :]).astype(jnp.float32)
    oh2 = (am2[:, None] == iota_e[None, :]).astype(jnp.float32)
    h_assign = oh1 + oh2
    it = jnp.arange(t, dtype=jnp.int32)
    tri = (it[:, None] > it[None, :]).astype(jnp.float32)
    cnt_upto = jax.lax.dot(tri, h_assign,
                           precision=lax.Precision.DEFAULT)
    counts = jnp.sum(h_assign, axis=0).astype(jnp.int32)
    padded = ((counts + SUB - 1) // SUB) * SUB
    pcum = jnp.cumsum(padded)
    pstart = pcum - padded
    base_rank0 = jnp.sum((pstart[None, :].astype(jnp.float32) + cnt_upto)
                         * oh1, axis=1)
    base_rank1 = jnp.sum((pstart[None, :].astype(jnp.float32) + cnt_upto)
                         * oh2, axis=1)
    dest = jnp.stack([base_rank0, base_rank1],
                     axis=1).astype(jnp.int32).reshape(-1)
    idx_pad = jnp.zeros((m_pad,), jnp.int32).at[dest].set(
        jnp.arange(m, dtype=jnp.int32) // K)

    gs = jnp.arange(n_blocks, dtype=jnp.int32) * SUB
    blk_expert = jnp.minimum(
        jnp.sum((gs[:, None] >= pcum[None, :]).astype(jnp.int32), axis=1),
        e - 1).astype(jnp.int32)
    ends = pstart + counts
    blk_end = jnp.sum(
        jnp.where(blk_expert[:, None] == iota_e[None, :], ends[None, :], 0),
        axis=1)
    blk_active = (gs < blk_end).astype(jnp.int32)

    x_sorted = hidden_states[idx_pad]
    y = _grouped_mlp(x_sorted, W13, W2, pstart.astype(jnp.int32), counts,
                     blk_expert, blk_active, e, d_ff)

    inv = dest.reshape(t, K)
    y2 = y[inv]
    out = jnp.sum(topk_w[:, :, None] * y2, axis=1)
    return out.astype(hidden_states.dtype)


# speedup vs baseline: 1.2237x; 1.2237x over previous
import functools

import jax
import jax.numpy as jnp
from jax import lax
from jax.experimental import pallas as pl
from jax.experimental.pallas import tpu as pltpu
from jax.experimental.pallas import tpu_sc as plsc

K = 2
SUB = 256
BF = 256


def _dispatch_sc(hidden_states, dest, m_pad):
    t, d_model = hidden_states.shape
    m = dest.shape[0]
    info = plsc.get_sparse_core_info()
    nw = info.num_cores * info.num_subcores
    b_per_w = m // nw
    ch = 64
    n_ch = b_per_w // ch
    dest2 = dest.reshape(nw * n_ch, ch)
    tok2 = (jnp.arange(m, dtype=jnp.int32) // K).reshape(nw * n_ch, ch)
    mesh = plsc.VectorSubcoreMesh(core_axis_name="c", subcore_axis_name="s")

    @functools.partial(
        pl.kernel, mesh=mesh,
        out_type=jax.ShapeDtypeStruct((m_pad, d_model), jnp.float32),
        scratch_types=[
            pltpu.VMEM((n_ch, ch), jnp.int32),
            pltpu.VMEM((n_ch, ch), jnp.int32),
            pltpu.VMEM((ch, d_model), jnp.float32),
            pltpu.SemaphoreType.DMA,
        ],
    )
    def dispatch(hidden_hbm, dest_hbm, tok_hbm, out_hbm,
                 dest_v, tok_v, rows_v, sem):
        wid = lax.axis_index("s") * info.num_cores + lax.axis_index("c")
        row0 = wid * n_ch
        pltpu.sync_copy(dest_hbm.at[pl.ds(row0, n_ch)], dest_v)
        pltpu.sync_copy(tok_hbm.at[pl.ds(row0, n_ch)], tok_v)
        for jc in range(n_ch):
            pltpu.async_copy(hidden_hbm.at[tok_v.at[jc]], rows_v, sem).wait()
            pltpu.async_copy(rows_v, out_hbm.at[dest_v.at[jc]], sem).wait()

    return dispatch(hidden_states, dest2, tok2)


def _gate_up_body(pstart_ref, counts_ref, x_ref, w1_ref, w3_ref, h_ref):
    e = pl.program_id(1)
    p0 = pstart_ref[e]
    cnt = counts_ref[e]
    nsub = (cnt + SUB - 1) // SUB
    w1 = w1_ref[0]
    w3 = w3_ref[0]

    def body(j, carry):
        off = pl.multiple_of(p0 + j * SUB, SUB)
        x = x_ref[pl.ds(off, SUB), :]
        gate = lax.dot_general(x, w1, (((1,), (1,)), ((), ())),
                               preferred_element_type=jnp.float32,
                               precision=lax.Precision.DEFAULT)
        up = lax.dot_general(x, w3, (((1,), (1,)), ((), ())),
                             preferred_element_type=jnp.float32,
                             precision=lax.Precision.DEFAULT)
        h = gate * jax.nn.sigmoid(gate) * up
        h_ref[pl.ds(off, SUB), :] = h.astype(jnp.bfloat16)
        return carry

    lax.fori_loop(0, nsub, body, 0)


def _down_body(be_ref, act_ref, h_ref, w2_ref, y_ref):
    g = pl.program_id(0)

    @pl.when(act_ref[g] > 0)
    def _():
        h = h_ref[...].astype(jnp.float32)
        y_ref[...] = lax.dot_general(
            h, w2_ref[0], (((1,), (1,)), ((), ())),
            preferred_element_type=jnp.float32,
            precision=lax.Precision.DEFAULT)


def _grouped_mlp(x_sorted, w13, w2, pstart, counts, blk_expert, blk_active,
                 n_experts, d_ff):
    m_pad, d_model = x_sorted.shape
    nf = d_ff // BF
    n_blocks = m_pad // SUB

    gu_spec = pltpu.PrefetchScalarGridSpec(
        num_scalar_prefetch=2,
        grid=(nf, n_experts),
        in_specs=[
            pl.BlockSpec((m_pad, d_model), lambda f, e, ps, ct: (0, 0)),
            pl.BlockSpec((1, BF, d_model), lambda f, e, ps, ct: (e, f, 0)),
            pl.BlockSpec((1, BF, d_model), lambda f, e, ps, ct: (e, nf + f, 0)),
        ],
        out_specs=pl.BlockSpec((m_pad, BF), lambda f, e, ps, ct: (0, f)),
    )
    h = pl.pallas_call(
        _gate_up_body,
        grid_spec=gu_spec,
        out_shape=jax.ShapeDtypeStruct((m_pad, d_ff), jnp.bfloat16),
        compiler_params=pltpu.CompilerParams(
            dimension_semantics=("arbitrary", "arbitrary"),
        ),
    )(pstart, counts, x_sorted, w13, w13)

    dn_spec = pltpu.PrefetchScalarGridSpec(
        num_scalar_prefetch=2,
        grid=(n_blocks,),
        in_specs=[
            pl.BlockSpec((SUB, d_ff), lambda g, be, act: (g, 0)),
            pl.BlockSpec((1, d_model, d_ff), lambda g, be, act: (be[g], 0, 0)),
        ],
        out_specs=pl.BlockSpec((SUB, d_model), lambda g, be, act: (g, 0)),
    )
    return pl.pallas_call(
        _down_body,
        grid_spec=dn_spec,
        out_shape=jax.ShapeDtypeStruct((m_pad, d_model), jnp.float32),
        compiler_params=pltpu.CompilerParams(
            dimension_semantics=("arbitrary",),
        ),
    )(blk_expert, blk_active, h, w2)


def kernel(hidden_states, use_grouped_topk, top_k, router_logits,
           renormalize, W13, W2):
    t, d_model = hidden_states.shape
    e = router_logits.shape[1]
    d_ff = W2.shape[2]
    m = t * K
    m_pad = m + e * SUB
    n_blocks = m_pad // SUB
    iota_e = jnp.arange(e, dtype=jnp.int32)

    logits = router_logits.astype(jnp.float32)
    mx = jnp.max(logits, axis=-1, keepdims=True)
    ex = jnp.exp(logits - mx)
    probs = ex / jnp.sum(ex, axis=-1, keepdims=True)
    m1 = jnp.max(probs, axis=-1)
    am1 = jnp.min(jnp.where(probs == m1[:, None], iota_e, e),
                  axis=-1).astype(jnp.int32)
    masked = jnp.where(iota_e[None, :] == am1[:, None], -jnp.inf, probs)
    m2 = jnp.max(masked, axis=-1)
    am2 = jnp.min(jnp.where(masked == m2[:, None], iota_e, e),
                  axis=-1).astype(jnp.int32)
    topk_w = jnp.stack([m1, m2], axis=1)
    denom = m1 + m2
    topk_w = jnp.where(jnp.asarray(renormalize), topk_w / denom[:, None],
                       topk_w)
    topk_w = topk_w * (jnp.asarray(1, jnp.float32)
                       - jnp.asarray(use_grouped_topk, jnp.float32))

    oh1 = (am1[:, None] == iota_e[None, :]).astype(jnp.float32)
    oh2 = (am2[:, None] == iota_e[None, :]).astype(jnp.float32)
    h_assign = oh1 + oh2
    it = jnp.arange(t, dtype=jnp.int32)
    tri = (it[:, None] > it[None, :]).astype(jnp.float32)
    cnt_upto = jax.lax.dot(tri, h_assign,
                           precision=lax.Precision.DEFAULT)
    counts = jnp.sum(h_assign, axis=0).astype(jnp.int32)
    padded = ((counts + SUB - 1) // SUB) * SUB
    pcum = jnp.cumsum(padded)
    pstart = pcum - padded
    base_rank0 = jnp.sum((pstart[None, :].astype(jnp.float32) + cnt_upto)
                         * oh1, axis=1)
    base_rank1 = jnp.sum((pstart[None, :].astype(jnp.float32) + cnt_upto)
                         * oh2, axis=1)
    dest = jnp.stack([base_rank0, base_rank1],
                     axis=1).astype(jnp.int32).reshape(-1)

    gs = jnp.arange(n_blocks, dtype=jnp.int32) * SUB
    blk_expert = jnp.minimum(
        jnp.sum((gs[:, None] >= pcum[None, :]).astype(jnp.int32), axis=1),
        e - 1).astype(jnp.int32)
    ends = pstart + counts
    blk_end = jnp.sum(
        jnp.where(blk_expert[:, None] == iota_e[None, :], ends[None, :], 0),
        axis=1)
    blk_active = (gs < blk_end).astype(jnp.int32)

    x_sorted = _dispatch_sc(hidden_states, dest, m_pad)
    y = _grouped_mlp(x_sorted, W13, W2, pstart.astype(jnp.int32), counts,
                     blk_expert, blk_active, e, d_ff)

    inv = dest.reshape(t, K)
    out = (topk_w[:, 0:1] * y[inv[:, 0]] + topk_w[:, 1:2] * y[inv[:, 1]])
    return out.astype(hidden_states.dtype)
